# baseline (speedup 1.0000x reference)
import functools

import jax
import jax.numpy as jnp
from jax import lax
from jax.experimental import pallas as pl
from jax.experimental.pallas import tpu as pltpu
from jax.experimental.pallas import tpu_sc as plsc

NUM_USERS = 100000
EMBED_DIM = 64
BATCH = 4096
CHUNK = 50048


@jax.jit
def _gather_t(user_idx, table_t):
    info = plsc.get_sparse_core_info()
    nw = info.num_cores * info.num_subcores
    rows_per_w = EMBED_DIM // nw
    sizes = (CHUNK, NUM_USERS - CHUNK)

    mesh = plsc.VectorSubcoreMesh(core_axis_name="c", subcore_axis_name="s")

    @functools.partial(
        pl.kernel,
        mesh=mesh,
        compiler_params=pltpu.CompilerParams(needs_layout_passes=False),
        out_type=jax.ShapeDtypeStruct((EMBED_DIM, BATCH), jnp.float32),
        scratch_types=[
            pltpu.VMEM((BATCH,), jnp.int32),
            pltpu.VMEM((CHUNK,), jnp.float32),
            pltpu.VMEM((NUM_USERS - CHUNK,), jnp.float32),
            pltpu.VMEM((BATCH,), jnp.float32),
            pltpu.VMEM((BATCH,), jnp.float32),
            pltpu.SemaphoreType.DMA,
            pltpu.SemaphoreType.DMA,
            pltpu.SemaphoreType.DMA,
            pltpu.SemaphoreType.DMA,
        ],
    )
    def k(idx_hbm, t_hbm, out_hbm, idx_v, buf0, buf1, orow0, orow1,
          sem_idx, sem0, sem1, sem_out):
        wid = lax.axis_index("s") * info.num_cores + lax.axis_index("c")
        idx_cp = pltpu.make_async_copy(idx_hbm, idx_v, sem_idx)
        idx_cp.start()

        bufs = (buf0, buf1)
        sems = (sem0, sem1)
        orows = (orow0, orow1)

        def half_dma(p, c):
            base = 0 if c == 0 else CHUNK
            return pltpu.make_async_copy(
                t_hbm.at[wid * rows_per_w + p, pl.ds(base, sizes[c])],
                bufs[c],
                sems[c],
            )

        half_dma(0, 0).start()
        half_dma(0, 1).start()
        idx_cp.wait()
        iota = lax.iota(jnp.int32, 16)

        for p in range(rows_per_w):
            half_dma(p, 0).wait()
            half_dma(p, 1).wait()
            orow = orows[p]
            for c in range(2):
                base = 0 if c == 0 else CHUNK
                n = sizes[c]
                buf = bufs[c]

                def gath(g, carry):
                    iv = idx_v[pl.ds(g * 16, 16)]
                    d = iv - base
                    m = d.astype(jnp.uint32) < jnp.uint32(n)
                    vals = plsc.load_gather(buf, [d], mask=m)
                    plsc.store_scatter(orow, [iota + g * 16], vals, mask=m)
                    return carry

                lax.fori_loop(0, BATCH // 16, gath, 0)
            if p + 1 < rows_per_w:
                half_dma(p + 1, 0).start()
                half_dma(p + 1, 1).start()
            pltpu.make_async_copy(
                orow, out_hbm.at[wid * rows_per_w + p], sem_out
            ).start()

        for p in range(rows_per_w):
            pltpu.make_async_copy(
                orows[p], out_hbm.at[wid * rows_per_w + p], sem_out
            ).wait()

    return k(user_idx, table_t)


def kernel(user_idx, table):
    out_t = _gather_t(user_idx.astype(jnp.int32), table.T)
    return out_t.T


# speedup vs baseline: 1.2449x; 1.2449x over previous
import functools

import jax
import jax.numpy as jnp
from jax import lax
from jax.experimental import pallas as pl
from jax.experimental.pallas import tpu as pltpu
from jax.experimental.pallas import tpu_sc as plsc

NUM_USERS = 100000
EMBED_DIM = 64
BATCH = 4096


@jax.jit
def _gather_t(user_idx, table_t):
    info = plsc.get_sparse_core_info()
    nw = info.num_cores * info.num_subcores
    rows_per_w = EMBED_DIM // nw

    mesh = plsc.VectorSubcoreMesh(core_axis_name="c", subcore_axis_name="s")

    @functools.partial(
        pl.kernel,
        mesh=mesh,
        compiler_params=pltpu.CompilerParams(needs_layout_passes=False),
        out_type=jax.ShapeDtypeStruct((EMBED_DIM, BATCH), jnp.float32),
        scratch_types=[
            pltpu.VMEM((BATCH,), jnp.int32),
            pltpu.VMEM((NUM_USERS,), jnp.float32),
            pltpu.VMEM((BATCH,), jnp.float32),
            pltpu.VMEM((BATCH,), jnp.float32),
            pltpu.SemaphoreType.DMA,
            pltpu.SemaphoreType.DMA,
            pltpu.SemaphoreType.DMA,
        ],
    )
    def k(idx_hbm, t_hbm, out_hbm, idx_v, row_v, orow0, orow1,
          sem_idx, sem_row, sem_out):
        wid = lax.axis_index("s") * info.num_cores + lax.axis_index("c")
        orows = (orow0, orow1)

        idx_cp = pltpu.make_async_copy(idx_hbm, idx_v, sem_idx)
        idx_cp.start()

        def row_dma(p):
            return pltpu.make_async_copy(
                t_hbm.at[wid * rows_per_w + p], row_v, sem_row
            )

        row_dma(0).start()
        idx_cp.wait()

        for p in range(rows_per_w):
            row_dma(p).wait()
            orow = orows[p]

            def gath(g, carry):
                iv = idx_v[pl.ds(g * 16, 16)]
                orow[pl.ds(g * 16, 16)] = plsc.load_gather(row_v, [iv])
                return carry

            lax.fori_loop(0, BATCH // 16, gath, 0)
            if p + 1 < rows_per_w:
                row_dma(p + 1).start()
            pltpu.make_async_copy(
                orow, out_hbm.at[wid * rows_per_w + p], sem_out
            ).start()

        for p in range(rows_per_w):
            pltpu.make_async_copy(
                orows[p], out_hbm.at[wid * rows_per_w + p], sem_out
            ).wait()

    return k(user_idx, table_t)


def kernel(user_idx, table):
    out_t = _gather_t(user_idx.astype(jnp.int32), table.T)
    return out_t.T
